# Initial kernel scaffold; baseline (speedup 1.0000x reference)
import jax
import jax.numpy as jnp
from jax.experimental import pallas as pl


def kernel(z, pos_edge_index, neg_edge_index):
    raise NotImplementedError("write your pallas kernel here")


# speedup vs baseline: 2.6874x; 2.6874x over previous
import functools

import jax
import jax.numpy as jnp
from jax import lax
from jax.experimental import pallas as pl
from jax.experimental.pallas import tpu as pltpu
from jax.experimental.pallas import tpu_sc as plsc

N_NODES = 10000
D_FEAT = 256
N_EDGES = 160000

NC = 2
NS = 16
NW = NC * NS
L = 16

C = 64
_CHUNKS_PER_SET = -(-N_EDGES // (NW * C))
EPW = _CHUNKS_PER_SET * C
EPAD = EPW * NW
M = 2 * EPAD
EPW2 = 2 * EPW
K2 = 2 * _CHUNKS_PER_SET


def _sc_body(z_hbm, src_hbm, dst_hbm, out_hbm,
             sidx, didx, s0, s1, d0, d1, lbuf, tbuf, sem0, sem1):
    wid = lax.axis_index("s") * NC + lax.axis_index("c")
    base = wid * EPW2

    pltpu.sync_copy(src_hbm.at[pl.ds(base, EPW2)], sidx)
    pltpu.sync_copy(dst_hbm.at[pl.ds(base, EPW2)], didx)

    bufs = ((s0, d0, sem0), (s1, d1, sem1))

    def issue(k, b):
        sb, db, sem = bufs[b]
        pltpu.make_async_copy(
            z_hbm.at[sidx.at[pl.ds(k * C, C)]], sb, sem).start()
        pltpu.make_async_copy(
            z_hbm.at[didx.at[pl.ds(k * C, C)]], db, sem).start()

    def wait(b):
        sb, db, sem = bufs[b]
        pltpu.make_async_copy(
            z_hbm.at[sidx.at[pl.ds(0, C)]], sb, sem).wait()
        pltpu.make_async_copy(
            z_hbm.at[didx.at[pl.ds(0, C)]], db, sem).wait()

    def compute(k, b):
        sb, db, _ = bufs[b]
        lane16 = lax.iota(jnp.int32, L) * L

        def group(q, carry):
            e0 = q * L
            for r in range(L):
                e = e0 + r
                accs = []
                for j in range(4):
                    a = (sb[e, pl.ds(j * 64, L)] * db[e, pl.ds(j * 64, L)])
                    for f in range(1, 4):
                        off = j * 64 + f * L
                        a = a + sb[e, pl.ds(off, L)] * db[e, pl.ds(off, L)]
                    accs.append(a)
                tbuf[pl.ds(r * L, L)] = (accs[0] + accs[1]) + (accs[2] + accs[3])
            vals = jnp.zeros((L,), jnp.float32)
            for col in range(L):
                vals = vals + plsc.load_gather(tbuf, [lane16 + col])
            lbuf[pl.ds(k * C + e0, L)] = vals
            return carry

        lax.fori_loop(0, C // L, group, 0)

    issue(0, 0)
    issue(1, 1)

    def outer(i, carry):
        g = i * 2
        for b in range(2):
            k = g + b
            wait(b)
            compute(k, b)

            @pl.when(k + 2 < K2)
            def _():
                issue(k + 2, b)
        return carry

    lax.fori_loop(0, K2 // 2, outer, 0)

    pltpu.sync_copy(lbuf, out_hbm.at[pl.ds(base, EPW2)])


_sc_gather_dot = functools.partial(
    pl.kernel,
    out_type=jax.ShapeDtypeStruct((M,), jnp.float32),
    mesh=plsc.VectorSubcoreMesh(core_axis_name="c", subcore_axis_name="s"),
    compiler_params=pltpu.CompilerParams(needs_layout_passes=False),
    scratch_types=[
        pltpu.VMEM((EPW2,), jnp.int32),
        pltpu.VMEM((EPW2,), jnp.int32),
        pltpu.VMEM((C, D_FEAT), jnp.float32),
        pltpu.VMEM((C, D_FEAT), jnp.float32),
        pltpu.VMEM((C, D_FEAT), jnp.float32),
        pltpu.VMEM((C, D_FEAT), jnp.float32),
        pltpu.VMEM((EPW2,), jnp.float32),
        pltpu.VMEM((L * L,), jnp.float32),
        pltpu.SemaphoreType.DMA,
        pltpu.SemaphoreType.DMA,
    ],
)(_sc_body)


def _loss_body(pos_ref, neg_ref, out_ref):
    eps = 1e-15
    x = pos_ref[...]
    s = 1.0 / (1.0 + jnp.exp(-x))
    pos_loss = -jnp.sum(jnp.log(s + eps)) / N_EDGES
    y = neg_ref[...]
    t = 1.0 / (1.0 + jnp.exp(-y))
    neg_loss = -jnp.sum(jnp.log(1.0 - t + eps)) / N_EDGES
    out_ref[0, 0] = pos_loss + neg_loss


_loss_reduce = pl.pallas_call(
    _loss_body,
    out_shape=jax.ShapeDtypeStruct((1, 1), jnp.float32),
    out_specs=pl.BlockSpec(memory_space=pltpu.SMEM),
)


def kernel(z, pos_edge_index, neg_edge_index):
    pad = jnp.zeros((EPAD - N_EDGES,), jnp.int32)
    srcs = jnp.concatenate(
        [pos_edge_index[0], pad, neg_edge_index[0], pad])
    dsts = jnp.concatenate(
        [pos_edge_index[1], pad, neg_edge_index[1], pad])
    logits = _sc_gather_dot(z, srcs, dsts)
    pos_logits = logits[:N_EDGES].reshape(1250, 128)
    neg_logits = logits[EPAD:EPAD + N_EDGES].reshape(1250, 128)
    loss = _loss_reduce(pos_logits, neg_logits)
    return loss[0, 0]
